# speedup vs baseline: 5.7042x; 5.7042x over previous
import functools

import jax
import jax.numpy as jnp
from jax import lax
from jax.experimental import pallas as pl
from jax.experimental.pallas import tpu as pltpu
from jax.experimental.pallas import tpu_sc as plsc

D = 128
NC = 2
NS = 16
NW = NC * NS
CHUNK = 512
LANES = 16


def _sc_body(pe_t_h, pe_h_h, pe_w_h, t_h, h_h, w_h, out_h,
             pt_v, ph_v, pw_v, ti_v, hi_v, wi_v, ob_v):
    wid = lax.axis_index("s") * NC + lax.axis_index("c")
    n = out_h.shape[0]
    per_w = n // NW
    base = wid * per_w

    pltpu.sync_copy(pe_t_h, pt_v)
    pltpu.sync_copy(pe_h_h, ph_v)
    pltpu.sync_copy(pe_w_h, pw_v)

    def chunk_body(i, carry):
        off = base + i * CHUNK
        pltpu.sync_copy(t_h.at[pl.ds(off, CHUNK)], ti_v)
        pltpu.sync_copy(h_h.at[pl.ds(off, CHUNK)], hi_v)
        pltpu.sync_copy(w_h.at[pl.ds(off, CHUNK)], wi_v)

        def group_body(g, gcarry):
            tids = ti_v[pl.ds(g * LANES, LANES)]
            hids = hi_v[pl.ds(g * LANES, LANES)]
            wids = wi_v[pl.ds(g * LANES, LANES)]
            for j in range(LANES):
                tr = tids[j]
                hr = hids[j]
                wr = wids[j]
                r = g * LANES + j
                for c in range(D // LANES):
                    v = (pt_v[tr, pl.ds(c * LANES, LANES)]
                         + ph_v[hr, pl.ds(c * LANES, LANES)]
                         + pw_v[wr, pl.ds(c * LANES, LANES)])
                    ob_v[r, pl.ds(c * LANES, LANES)] = v
            return gcarry

        lax.fori_loop(0, CHUNK // LANES, group_body, 0)
        pltpu.sync_copy(ob_v, out_h.at[pl.ds(off, CHUNK)])
        return carry

    lax.fori_loop(0, per_w // CHUNK, chunk_body, 0)


def kernel(pe_t, pe_h, pe_w, t, h, w):
    b, l = t.shape
    n = b * l
    tf = t.reshape(n)
    hf = h.reshape(n)
    wf = w.reshape(n)
    mesh = plsc.VectorSubcoreMesh(core_axis_name="c", subcore_axis_name="s")
    run = pl.kernel(
        _sc_body,
        mesh=mesh,
        out_type=jax.ShapeDtypeStruct((n, D), jnp.float32),
        scratch_types=[
            pltpu.VMEM((pe_t.shape[0], D), jnp.float32),
            pltpu.VMEM((pe_h.shape[0], D), jnp.float32),
            pltpu.VMEM((pe_w.shape[0], D), jnp.float32),
            pltpu.VMEM((CHUNK,), jnp.int32),
            pltpu.VMEM((CHUNK,), jnp.int32),
            pltpu.VMEM((CHUNK,), jnp.int32),
            pltpu.VMEM((CHUNK, D), jnp.float32),
        ],
    )
    out = run(pe_t, pe_h, pe_w, tf, hf, wf)
    return out.reshape(b, l, D)
